# baseline (speedup 1.0000x reference)
import jax
import jax.numpy as jnp
from jax.experimental import pallas as pl
from jax.experimental.pallas import tpu as pltpu

QS = 16384
NB_ROWS = 4096
ROW = 16 * 8 * 8
TAIL = QS - NB_ROWS


def _store_kernel(bf, f, bl, lab, out, lab_out, sem):
    copies = (
        pltpu.make_async_copy(bf, out.at[pl.ds(0, NB_ROWS)], sem.at[0]),
        pltpu.make_async_copy(
            f.at[pl.ds(NB_ROWS, TAIL)], out.at[pl.ds(NB_ROWS, TAIL)], sem.at[1]
        ),
        pltpu.make_async_copy(bl, lab_out.at[pl.ds(0, NB_ROWS)], sem.at[2]),
        pltpu.make_async_copy(
            lab.at[pl.ds(NB_ROWS, TAIL)], lab_out.at[pl.ds(NB_ROWS, TAIL)], sem.at[3]
        ),
    )
    for c in copies:
        c.start()
    for c in copies:
        c.wait()


def kernel(batch_features, batch_labels, features, labels):
    bf = batch_features.reshape(NB_ROWS, ROW)
    f = features.reshape(QS, ROW)
    out, lab_out = pl.pallas_call(
        _store_kernel,
        in_specs=[pl.BlockSpec(memory_space=pltpu.MemorySpace.HBM)] * 4,
        out_specs=[pl.BlockSpec(memory_space=pltpu.MemorySpace.HBM)] * 2,
        out_shape=[
            jax.ShapeDtypeStruct((QS, ROW), jnp.float32),
            jax.ShapeDtypeStruct((QS,), jnp.int32),
        ],
        scratch_shapes=[pltpu.SemaphoreType.DMA((4,))],
    )(bf, f, batch_labels, labels)
    return out.reshape(QS, 16, 8, 8), lab_out


# speedup vs baseline: 1.0001x; 1.0001x over previous
import jax
import jax.numpy as jnp
from jax.experimental import pallas as pl
from jax.experimental.pallas import tpu as pltpu

QS = 16384
NB_ROWS = 4096
ROW = 16 * 8 * 8
TAIL = QS - NB_ROWS


CHUNK = 1024


def _store_kernel(bf, f, bl, lab, out, lab_out, sem):
    copies = []
    for j in range(NB_ROWS // CHUNK):
        copies.append(
            pltpu.make_async_copy(
                bf.at[pl.ds(j * CHUNK, CHUNK)],
                out.at[pl.ds(j * CHUNK, CHUNK)],
                sem.at[len(copies)],
            )
        )
    for j in range(TAIL // CHUNK):
        base = NB_ROWS + j * CHUNK
        copies.append(
            pltpu.make_async_copy(
                f.at[pl.ds(base, CHUNK)],
                out.at[pl.ds(base, CHUNK)],
                sem.at[len(copies)],
            )
        )
    copies.append(
        pltpu.make_async_copy(bl, lab_out.at[pl.ds(0, NB_ROWS)], sem.at[len(copies)])
    )
    copies.append(
        pltpu.make_async_copy(
            lab.at[pl.ds(NB_ROWS, TAIL)],
            lab_out.at[pl.ds(NB_ROWS, TAIL)],
            sem.at[len(copies)],
        )
    )
    for c in copies:
        c.start()
    for c in copies:
        c.wait()


N_SEM = QS // CHUNK + 2


def kernel(batch_features, batch_labels, features, labels):
    bf = batch_features.reshape(NB_ROWS, ROW)
    f = features.reshape(QS, ROW)
    out, lab_out = pl.pallas_call(
        _store_kernel,
        in_specs=[pl.BlockSpec(memory_space=pltpu.MemorySpace.HBM)] * 4,
        out_specs=[pl.BlockSpec(memory_space=pltpu.MemorySpace.HBM)] * 2,
        out_shape=[
            jax.ShapeDtypeStruct((QS, ROW), jnp.float32),
            jax.ShapeDtypeStruct((QS,), jnp.int32),
        ],
        scratch_shapes=[pltpu.SemaphoreType.DMA((N_SEM,))],
    )(bf, f, batch_labels, labels)
    return out.reshape(QS, 16, 8, 8), lab_out


# speedup vs baseline: 12.2694x; 12.2679x over previous
import jax
import jax.numpy as jnp
from jax.experimental import pallas as pl

QS = 16384
NB_ROWS = 4096
ROW = 16 * 8 * 8
BLK = 2048
GRID = QS // BLK
NBB = NB_ROWS // BLK


def _store_kernel(batch_ref, feat_ref, blab_ref, lab_ref, out_ref, lab_out_ref):
    i = pl.program_id(0)

    @pl.when(i < NBB)
    def _():
        out_ref[...] = batch_ref[...]

    @pl.when(i >= NBB)
    def _():
        out_ref[...] = feat_ref[...]

    @pl.when(i == 0)
    def _():
        lab_out_ref[0:32, :] = blab_ref[...]
        lab_out_ref[32:, :] = lab_ref[32:, :]


def kernel(batch_features, batch_labels, features, labels):
    bf = batch_features.reshape(NB_ROWS, ROW)
    f = features.reshape(QS, ROW)
    bl = batch_labels.reshape(32, 128)
    lab = labels.reshape(128, 128)
    out, lab_out = pl.pallas_call(
        _store_kernel,
        grid=(GRID,),
        in_specs=[
            pl.BlockSpec((BLK, ROW), lambda i: (jnp.minimum(i, NBB - 1), 0)),
            pl.BlockSpec((BLK, ROW), lambda i: (jnp.maximum(i, NBB), 0)),
            pl.BlockSpec((32, 128), lambda i: (0, 0)),
            pl.BlockSpec((128, 128), lambda i: (0, 0)),
        ],
        out_specs=[
            pl.BlockSpec((BLK, ROW), lambda i: (i, 0)),
            pl.BlockSpec((128, 128), lambda i: (0, 0)),
        ],
        out_shape=[
            jax.ShapeDtypeStruct((QS, ROW), jnp.float32),
            jax.ShapeDtypeStruct((128, 128), jnp.int32),
        ],
    )(bf, f, bl, lab)
    return out.reshape(QS, 16, 8, 8), lab_out.reshape(QS)
